# baseline (speedup 1.0000x reference)
import functools

import jax
import jax.numpy as jnp
from jax import lax
from jax.experimental import pallas as pl
from jax.experimental.pallas import tpu as pltpu
from jax.experimental.pallas import tpu_sc as plsc

VOCAB = 28996
EMBED_DIM = 64
NSEQ = 4096
SEQLEN = 200

NC, NS, L = 2, 16, 16
NW = NC * NS

BPW = NSEQ // NW

NBUF = 4
N_GROUPS = SEQLEN // NBUF


@functools.cache
def _build_gather_kernel():
    mesh = plsc.VectorSubcoreMesh(core_axis_name="c", subcore_axis_name="s")
    return functools.partial(
        pl.kernel,
        out_type=jax.ShapeDtypeStruct((SEQLEN, EMBED_DIM, NSEQ), jnp.float32),
        mesh=mesh,
        compiler_params=pltpu.CompilerParams(needs_layout_passes=False),
        scratch_types=[
            pltpu.VMEM((SEQLEN, BPW), jnp.int32),
            pltpu.VMEM((NBUF, BPW, 128), jnp.float32),
            pltpu.VMEM((NBUF, EMBED_DIM, BPW), jnp.float32),
            [pltpu.SemaphoreType.DMA] * NBUF,
            [pltpu.SemaphoreType.DMA] * NBUF,
        ],
    )(_gather_body)


def _gather_body(xt_hbm, tab_hbm, out_hbm, idx_v, rows_r, tb_r, gsems, osems):
    wid = lax.axis_index("s") * NC + lax.axis_index("c")

    pltpu.sync_copy(xt_hbm.at[wid], idx_v)

    iota = lax.iota(jnp.int32, 16)
    colmod = [(iota + d) & 15 for d in range(16)]

    def gather(t, b):
        return pltpu.make_async_copy(
            tab_hbm.at[idx_v.at[t]], rows_r.at[b], gsems[b]
        )

    def out_dma(t, b):
        return pltpu.make_async_copy(
            tb_r.at[b],
            out_hbm.at[t, :, pl.ds(wid * BPW, BPW)],
            osems[b],
        )

    def transpose(b):
        def blk_body(blk, carry):
            l0 = (blk >> 2) * 16
            e0 = (blk & 3) * 16
            rowv = iota + l0
            for d in range(16):
                col = colmod[d] + e0
                v = plsc.load_gather(rows_r.at[b], [rowv, col])
                plsc.store_scatter(tb_r.at[b], [col, rowv], v)
            return carry

        lax.fori_loop(0, 32, blk_body, 0, unroll=4)

    def group(g, carry):
        for b in range(NBUF):
            t = g * NBUF + b

            @pl.when(g > 0)
            def _wait_out():
                out_dma(t - NBUF, b).wait()

            gather(t, b).start()
        for b in range(NBUF):
            t = g * NBUF + b
            gather(t, b).wait()
            transpose(b)
            out_dma(t, b).start()
        return carry

    lax.fori_loop(0, N_GROUPS, group, 0)

    for b in range(NBUF):
        out_dma((N_GROUPS - 1) * NBUF + b, b).wait()


def kernel(x, table):
    xt = x.T.reshape(SEQLEN, NW, BPW).transpose(1, 0, 2)
    flat = table.reshape(-1)
    flat_pad = jnp.concatenate([flat, jnp.zeros((EMBED_DIM,), jnp.float32)])
    shifted = flat_pad[EMBED_DIM:].reshape(VOCAB, EMBED_DIM)
    tab3 = jnp.concatenate([table, shifted], axis=1)
    out_t = _build_gather_kernel()(xt, tab3)
    return jnp.transpose(out_t, (2, 0, 1))


# speedup vs baseline: 1.1801x; 1.1801x over previous
import functools

import jax
import jax.numpy as jnp
from jax import lax
from jax.experimental import pallas as pl
from jax.experimental.pallas import tpu as pltpu
from jax.experimental.pallas import tpu_sc as plsc

VOCAB = 28996
EMBED_DIM = 64
NSEQ = 4096
SEQLEN = 200

NC, NS, L = 2, 16, 16
NW = NC * NS

BPW = NSEQ // NW

NBUF = 4
N_GROUPS = SEQLEN // NBUF


@functools.cache
def _build_gather_kernel():
    mesh = plsc.VectorSubcoreMesh(core_axis_name="c", subcore_axis_name="s")
    return functools.partial(
        pl.kernel,
        out_type=jax.ShapeDtypeStruct((SEQLEN, EMBED_DIM, NSEQ), jnp.float32),
        mesh=mesh,
        compiler_params=pltpu.CompilerParams(needs_layout_passes=False),
        scratch_types=[
            pltpu.VMEM((SEQLEN, BPW), jnp.int32),
            pltpu.VMEM((NBUF, BPW, 128), jnp.float32),
            pltpu.VMEM((NBUF, EMBED_DIM, BPW), jnp.float32),
            [pltpu.SemaphoreType.DMA] * NBUF,
            [pltpu.SemaphoreType.DMA] * NBUF,
        ],
    )(_gather_body)


def _gather_body(xt_hbm, tab_hbm, out_hbm, idx_v, rows_r, tb_r, gsems, osems):
    wid = lax.axis_index("s") * NC + lax.axis_index("c")

    pltpu.sync_copy(xt_hbm.at[wid], idx_v)

    iota = lax.iota(jnp.int32, 16)
    colmod = [(iota + d) & 15 for d in range(16)]

    def gather(t, b):
        return pltpu.make_async_copy(
            tab_hbm.at[idx_v.at[t]], rows_r.at[b], gsems[b]
        )

    def out_dma(t, b):
        return pltpu.make_async_copy(
            tb_r.at[b],
            out_hbm.at[t, :, pl.ds(wid * BPW, BPW)],
            osems[b],
        )

    def transpose(b):
        def blk_body(blk, carry):
            l0 = (blk >> 2) * 16
            e0 = (blk & 3) * 16
            rowv = iota + l0
            for d in range(16):
                col = colmod[d] + e0
                v = plsc.load_gather(rows_r.at[b], [rowv, col])
                plsc.store_scatter(tb_r.at[b], [col, rowv], v)
            return carry

        lax.fori_loop(0, 32, blk_body, 0, unroll=2)

    def group(g, carry):
        for b in range(NBUF):
            t = g * NBUF + b

            @pl.when(g > 0)
            def _wait_out():
                out_dma(t - NBUF, b).wait()

            gather(t, b).start()
        for b in range(NBUF):
            t = g * NBUF + b
            gather(t, b).wait()
            transpose(b)
            out_dma(t, b).start()
        return carry

    lax.fori_loop(0, N_GROUPS, group, 0)

    for b in range(NBUF):
        out_dma((N_GROUPS - 1) * NBUF + b, b).wait()


def kernel(x, table):
    xt = x.T.reshape(SEQLEN, NW, BPW).transpose(1, 0, 2)
    flat = table.reshape(-1)
    flat_pad = jnp.concatenate([flat, jnp.zeros((EMBED_DIM,), jnp.float32)])
    shifted = flat_pad[EMBED_DIM:].reshape(VOCAB, EMBED_DIM)
    tab3 = jnp.concatenate([table, shifted], axis=1)
    out_t = _build_gather_kernel()(xt, tab3)
    return jnp.transpose(out_t, (2, 0, 1))
